# baseline (speedup 1.0000x reference)
import dataclasses
import functools

import jax
import jax.numpy as jnp
from jax import lax
from jax.experimental import pallas as pl
from jax.experimental.pallas import tpu as pltpu
from jax.experimental.pallas import tpu_sc as plsc

N = 10000
E = 320000
D = 128
H = 128
T = 112
EPS_MSG = 1e-7
EPS_BN = 1e-5

NC = 2
NS = 16
NW = NC * NS
NN = 10240
NBW = NN // NW
EH = E // NC
EWS = EH // NS
CHK = 2000
NVC = CHK // 16
BB = 1024
CE = 2048
GB = 128
SB = 128
ESH = EH + CE + 16
RPS = NN + 16

I32 = jnp.int32
F32 = jnp.float32
F16 = jnp.float16
I16 = jnp.int16

_SC_MESH = plsc.VectorSubcoreMesh(core_axis_name="c", subcore_axis_name="s")
_CP = pltpu.CompilerParams()
if "needs_layout_passes" in pltpu.CompilerParams.__dataclass_fields__:
    _CP = dataclasses.replace(_CP, needs_layout_passes=False)

_PREC = lax.Precision.DEFAULT


def _mm(a, b):
    return lax.dot_general(a, b, (((1,), (0,)), ((), ())),
                           precision=_PREC, preferred_element_type=F32)


def _bn_relu(h, gamma, beta):
    mu = jnp.mean(h, axis=0, keepdims=True)
    var = jnp.mean((h - mu) ** 2, axis=0, keepdims=True)
    return jnp.maximum(gamma * (h - mu) / jnp.sqrt(var + EPS_BN) + beta, 0.0)


def _encode_body(x_ref, w_ref, b_ref, h_ref, m_ref):
    h = _mm(x_ref[...], w_ref[...]) + b_ref[...]
    h_ref[...] = h
    m_ref[...] = jnp.maximum(h, 0.0) + EPS_MSG


def _encode(x, w, b):
    return pl.pallas_call(
        _encode_body,
        out_shape=(jax.ShapeDtypeStruct((N, H), F32),
                   jax.ShapeDtypeStruct((N, H), F32)),
    )(x, w, b.reshape(1, H))


def _layer_body(has_res, t_ref, h2_ref, agg_ref, w_ref, b_ref, g_ref, bt_ref,
                tn_ref, h2n_ref, mn_ref):
    agg = agg_ref[...][:N].astype(F32)
    t = _mm(h2_ref[...] + agg, w_ref[...]) + b_ref[...]
    if has_res:
        t = t + t_ref[...]
    h2 = _bn_relu(t, g_ref[...], bt_ref[...])
    tn_ref[...] = t
    h2n_ref[...] = h2
    mn_ref[...] = h2 + EPS_MSG


def _layer(t, h2, agg, w, b, g, bt, has_res):
    return pl.pallas_call(
        functools.partial(_layer_body, has_res),
        out_shape=(jax.ShapeDtypeStruct((N, H), F32),
                   jax.ShapeDtypeStruct((N, H), F32),
                   jax.ShapeDtypeStruct((N, H), F32)),
    )(t, h2, agg, w, b.reshape(1, H), g.reshape(1, H), bt.reshape(1, H))


def _final_body(t_ref, h2_ref, agg_ref, w_ref, b_ref, g_ref, bt_ref,
                wp_ref, bp_ref, o_ref):
    agg = agg_ref[...][:N].astype(F32)
    t = _mm(h2_ref[...] + agg, w_ref[...]) + b_ref[...] + t_ref[...]
    h = _bn_relu(t, g_ref[...], bt_ref[...])
    o_ref[...] = _mm(h, wp_ref[...]) + bp_ref[...]


def _final(t, h2, agg, w, b, g, bt, wp, bp):
    return pl.pallas_call(
        _final_body,
        out_shape=jax.ShapeDtypeStruct((N, T), F32),
    )(t, h2, agg, w, b.reshape(1, H), g.reshape(1, H), bt.reshape(1, H),
      wp, bp.reshape(1, T))


def _sort_body(src_hbm, dst_hbm, srt_hbm, rp_hbm,
               cnt_my, off_my, dbuf, sbuf, posb, cbuf, csh, srt_sh, sem):
    c = lax.axis_index("c")
    s = lax.axis_index("s")
    base_e = c * EH + s * EWS

    def zero_body(i, _):
        cnt_my[pl.ds(i * 16, 16)] = jnp.zeros((16,), I32)
        return 0
    lax.fori_loop(0, NN // 16, zero_body, 0)

    for k in range(EWS // CHK):
        pltpu.sync_copy(dst_hbm.at[pl.ds(base_e + k * CHK, CHK)],
                        dbuf.at[pl.ds(0, CHK)])

        def hist_body(i, _):
            v = dbuf[pl.ds(i * 16, 16)]
            cnt, last = plsc.scan_count(v)
            cur = plsc.load_gather(cnt_my, [v])
            plsc.store_scatter(cnt_my, [v], cur + cnt, mask=last)
            return 0
        lax.fori_loop(0, NVC, hist_body, 0)

    pltpu.sync_copy(cnt_my, csh.at[s])
    plsc.subcore_barrier()

    def blk_body(blk, carry):
        for r in range(NS):
            pltpu.sync_copy(csh.at[r, pl.ds(blk * BB, BB)], cbuf.at[r])

        def vreg_body(j, carry):
            t = jnp.zeros((16,), I32)
            p = jnp.zeros((16,), I32)
            for r in range(NS):
                row = cbuf[r, pl.ds(j * 16, 16)]
                t = t + row
                p = p + jnp.where(r < s, row, jnp.zeros((16,), I32))
            incl = plsc.cumsum(t)
            excl = incl - t + carry
            cnt_my[pl.ds(blk * BB + j * 16, 16)] = excl
            off_my[pl.ds(blk * BB + j * 16, 16)] = excl + p
            return carry + jnp.max(incl, axis=0)
        return lax.fori_loop(0, BB // 16, vreg_body, carry)
    lax.fori_loop(0, NN // BB, blk_body, jnp.zeros((), I32))

    nb = NN // NS
    pltpu.sync_copy(cnt_my.at[pl.ds(s * nb, nb)],
                    rp_hbm.at[pl.ds(c * RPS + s * nb, nb)])

    @pl.when(s == NS - 1)
    def _():
        dbuf[pl.ds(0, 16)] = jnp.full((16,), EH, I32)
        pltpu.sync_copy(dbuf.at[pl.ds(0, 16)], rp_hbm.at[pl.ds(c * RPS + NN, 16)])

    dump = EH + 8 + s * 8
    for k in range(EWS // CHK):
        pltpu.sync_copy(dst_hbm.at[pl.ds(base_e + k * CHK, CHK)],
                        dbuf.at[pl.ds(0, CHK)])
        pltpu.sync_copy(src_hbm.at[pl.ds(base_e + k * CHK, CHK)],
                        sbuf.at[pl.ds(0, CHK)])
        for j in range(16):
            for u in range(8):
                eb = j * SB + u * 16
                if eb < CHK:
                    v = dbuf[pl.ds(eb, 16)]
                    cnt, last = plsc.scan_count(v)
                    off = plsc.load_gather(off_my, [v])
                    posb[0, j, pl.ds(u * 16, 16)] = off + cnt - 1
                    plsc.store_scatter(off_my, [v], off + cnt, mask=last)
                else:
                    posb[0, j, pl.ds(u * 16, 16)] = jnp.full((16,), dump, I32)
        copies = []
        for j in range(16):
            copies.append(pltpu.async_copy(
                sbuf.at[pl.ds(j * SB, SB)], srt_sh.at[posb.at[0, j]], sem))
        for cp in copies:
            cp.wait()

    plsc.subcore_barrier()
    ews = EH // NS
    pltpu.sync_copy(srt_sh.at[pl.ds(s * ews, ews)],
                    cnt_my.at[pl.ds(0, ews)])
    pltpu.sync_copy(cnt_my.at[pl.ds(0, ews)],
                    srt_hbm.at[pl.ds(c * ESH + s * ews, ews)])


def _sort_edges(src, dst):
    k = pl.kernel(
        _sort_body,
        out_type=(jax.ShapeDtypeStruct((NC * ESH,), I32),
                  jax.ShapeDtypeStruct((NC * RPS,), I32)),
        mesh=_SC_MESH,
        compiler_params=_CP,
        scratch_types=[
            pltpu.VMEM((NN,), I32),
            pltpu.VMEM((NN,), I32),
            pltpu.VMEM((CHK + 48,), I32),
            pltpu.VMEM((CHK + 48,), I32),
            pltpu.VMEM((1, 16, SB), I32),
            pltpu.VMEM((NS, BB), I32),
            pltpu.VMEM_SHARED((NS, NN), I32),
            pltpu.VMEM_SHARED((EH + 144,), I32),
            pltpu.SemaphoreType.DMA,
        ],
    )
    return k(src, dst)


def _segmax_body(m_hbm, srt_hbm, rp_hbm, agg_hbm,
                 aggb, rpb, ib, ib2, rows0, rows1, sem0, sem1):
    c = lax.axis_index("c")
    s = lax.axis_index("s")
    w = s * NC + c
    node_base = w * NBW


    def zrow(r, _):
        for u in range(8):
            aggb[r, pl.ds(u * 16, 16)] = jnp.zeros((16,), F32)
        return 0
    lax.fori_loop(0, NBW, zrow, 0)

    zacc = (jnp.zeros((16,), F32),) * 8

    for run in range(NC):
        rp_off = run * RPS + node_base
        pltpu.sync_copy(rp_hbm.at[pl.ds(rp_off, NBW + 16)],
                        rpb.at[pl.ds(0, NBW + 16)])

        def getrp(n):
            return rpb[pl.ds(n, 16)][0]

        e0 = getrp(0)
        e1 = getrp(NBW)
        count = e1 - e0

        def deg(n):
            return rpb[pl.ds(n + 1, 16)][0] - rpb[pl.ds(n, 16)][0]

        def adv_cond(cur):
            return jnp.logical_and(cur < NBW, deg(cur) == 0)

        def adv_body(cur):
            return cur + 1
        cur0 = lax.while_loop(adv_cond, adv_body, jnp.zeros((), I32))
        rem0 = jnp.where(cur0 < NBW, deg(jnp.minimum(cur0, NBW - 1)),
                         jnp.zeros((), I32))

        nchunks = (count + CE - 1) // CE

        def chunk_body(ck, st):
            cur, rem, acc = st
            start = e0 + ck * CE
            clen = jnp.minimum(CE, count - ck * CE)
            offa = (start // 8) * 8
            pad = start - offa
            pltpu.sync_copy(srt_hbm.at[pl.ds(run * ESH + offa, CE + 16)], ib)

            def clampb(i, _):
                v = ib[pl.ds(pad + i * 16, 16)]
                ib2[pl.ds(i * 16, 16)] = jnp.clip(v, 0, N - 1)
                return 0
            lax.fori_loop(0, CE // 16, clampb, 0)

            nb = (clen + GB - 1) // GB

            def fire(b, rbuf, rsem):
                return pltpu.async_copy(
                    m_hbm.at[ib2.at[pl.ds(b * GB, GB)]], rbuf, rsem)

            cp0 = fire(jnp.zeros((), I32), rows0, sem0)

            def batch_body(b, st):
                even = (b % 2) == 0

                @pl.when(jnp.logical_and(even, b + 1 < nb))
                def _():
                    fire(b + 1, rows1, sem1)

                @pl.when(jnp.logical_and(~even, b + 1 < nb))
                def _():
                    fire(b + 1, rows0, sem0)

                blen = jnp.minimum(GB, clen - b * GB)

                def proc(rbuf, rsem, st):
                    pltpu.make_async_copy(
                        m_hbm.at[ib2.at[pl.ds(0, GB)]], rbuf, rsem).wait()

                    def piece_cond(pst):
                        e, cur, rem, acc = pst
                        return jnp.logical_and(e < blen, cur < NBW)

                    def piece_body(pst):
                        e, cur, rem, acc = pst
                        t = jnp.minimum(rem, blen - e)

                        def maxbody2(i, acc):
                            b0 = e + i * 2
                            a = tuple(
                                jnp.maximum(acc[u], rbuf[b0, pl.ds(u * 16, 16)])
                                for u in range(8))
                            return tuple(
                                jnp.maximum(a[u], rbuf[b0 + 1, pl.ds(u * 16, 16)])
                                for u in range(8))
                        acc = lax.fori_loop(0, t // 2, maxbody2, acc)
                        acc = lax.cond(
                            t % 2 == 1,
                            lambda a: tuple(
                                jnp.maximum(a[u],
                                            rbuf[e + t - 1, pl.ds(u * 16, 16)])
                                for u in range(8)),
                            lambda a: a, acc)
                        rem = rem - t
                        e = e + t

                        def flush(op):
                            cur, acc = op
                            for u in range(8):
                                old = aggb[cur, pl.ds(u * 16, 16)]
                                aggb[cur, pl.ds(u * 16, 16)] = (
                                    jnp.maximum(old, acc[u]))
                            nxt = lax.while_loop(adv_cond, adv_body, cur + 1)
                            nrem = jnp.where(nxt < NBW,
                                             deg(jnp.minimum(nxt, NBW - 1)),
                                             jnp.zeros((), I32))
                            return nxt, nrem, zacc

                        def noflush(op):
                            cur, acc = op
                            return cur, rem, acc

                        cur, rem, acc = lax.cond(rem == 0, flush, noflush,
                                                 (cur, acc))
                        return e, cur, rem, acc

                    cur0b, rem0b, acc0b = st
                    _, cur1, rem1, acc1 = lax.while_loop(
                        piece_cond, piece_body,
                        (jnp.zeros((), I32), cur0b, rem0b, acc0b))
                    return cur1, rem1, acc1

                return lax.cond(even,
                                lambda st: proc(rows0, sem0, st),
                                lambda st: proc(rows1, sem1, st), st)

            cur, rem, acc = lax.fori_loop(0, nb, batch_body, (cur, rem, acc))
            return cur, rem, acc

        lax.fori_loop(0, nchunks, chunk_body, (cur0, rem0, zacc))

    pltpu.sync_copy(aggb, agg_hbm.at[pl.ds(node_base, NBW)])


def _segmax(m, srt, rp):
    k = pl.kernel(
        _segmax_body,
        out_type=jax.ShapeDtypeStruct((NN, H), F32),
        mesh=_SC_MESH,
        compiler_params=_CP,
        scratch_types=[
            pltpu.VMEM((NBW, H), F32),
            pltpu.VMEM((NBW + 32,), I32),
            pltpu.VMEM((CE + 16,), I32),
            pltpu.VMEM((CE + 16,), I32),
            pltpu.VMEM((GB, H), F32),
            pltpu.VMEM((GB, H), F32),
            pltpu.SemaphoreType.DMA,
            pltpu.SemaphoreType.DMA,
        ],
    )
    return k(m, srt, rp)


def kernel(x, edge_index, batch, W_enc, b_enc, gcn_W, gcn_b, bn_gamma,
           bn_beta, W_pred, b_pred):
    src = edge_index[0]
    dst = edge_index[1]
    srt, rp = _sort_edges(src, dst)
    h0, m0 = _encode(x, W_enc, b_enc)
    a0 = _segmax(m0, srt, rp)
    t1, h2_1, m1 = _layer(h0, h0, a0, gcn_W[0], gcn_b[0],
                          bn_gamma[0], bn_beta[0], has_res=False)
    a1 = _segmax(m1, srt, rp)
    t2, h2_2, m2 = _layer(t1, h2_1, a1, gcn_W[1], gcn_b[1],
                          bn_gamma[1], bn_beta[1], has_res=True)
    a2 = _segmax(m2, srt, rp)
    return _final(t2, h2_2, a2, gcn_W[2], gcn_b[2], bn_gamma[2], bn_beta[2],
                  W_pred, b_pred)


# speedup vs baseline: 1.0467x; 1.0467x over previous
import dataclasses
import functools

import jax
import jax.numpy as jnp
from jax import lax
from jax.experimental import pallas as pl
from jax.experimental.pallas import tpu as pltpu
from jax.experimental.pallas import tpu_sc as plsc

N = 10000
E = 320000
D = 128
H = 128
T = 112
EPS_MSG = 1e-7
EPS_BN = 1e-5

NC = 2
NS = 16
NW = NC * NS
NN = 10240
NBW = NN // NW
EH = E // NC
EWS = EH // NS
CHK = 2000
NVC = CHK // 16
BB = 1024
CE = 2048
GB = 128
SB = 128
ESH = EH + CE + 16
RPS = NN + 16

I32 = jnp.int32
F32 = jnp.float32
F16 = jnp.float16
I16 = jnp.int16

_SC_MESH = plsc.VectorSubcoreMesh(core_axis_name="c", subcore_axis_name="s")
_CP = pltpu.CompilerParams()
if "needs_layout_passes" in pltpu.CompilerParams.__dataclass_fields__:
    _CP = dataclasses.replace(_CP, needs_layout_passes=False)

_PREC = lax.Precision.DEFAULT


def _mm(a, b):
    return lax.dot_general(a, b, (((1,), (0,)), ((), ())),
                           precision=_PREC, preferred_element_type=F32)


def _bn_relu(h, gamma, beta):
    mu = jnp.mean(h, axis=0, keepdims=True)
    var = jnp.mean((h - mu) ** 2, axis=0, keepdims=True)
    return jnp.maximum(gamma * (h - mu) / jnp.sqrt(var + EPS_BN) + beta, 0.0)


def _encode_body(x_ref, w_ref, b_ref, h_ref, m_ref):
    h = _mm(x_ref[...], w_ref[...]) + b_ref[...]
    h_ref[...] = h
    m_ref[...] = jnp.maximum(h, 0.0) + EPS_MSG


def _encode(x, w, b):
    return pl.pallas_call(
        _encode_body,
        out_shape=(jax.ShapeDtypeStruct((N, H), F32),
                   jax.ShapeDtypeStruct((N, H), F32)),
    )(x, w, b.reshape(1, H))


def _layer_body(has_res, t_ref, h2_ref, agg_ref, w_ref, b_ref, g_ref, bt_ref,
                tn_ref, h2n_ref, mn_ref):
    agg = agg_ref[...][:N].astype(F32)
    t = _mm(h2_ref[...] + agg, w_ref[...]) + b_ref[...]
    if has_res:
        t = t + t_ref[...]
    h2 = _bn_relu(t, g_ref[...], bt_ref[...])
    tn_ref[...] = t
    h2n_ref[...] = h2
    mn_ref[...] = h2 + EPS_MSG


def _layer(t, h2, agg, w, b, g, bt, has_res):
    return pl.pallas_call(
        functools.partial(_layer_body, has_res),
        out_shape=(jax.ShapeDtypeStruct((N, H), F32),
                   jax.ShapeDtypeStruct((N, H), F32),
                   jax.ShapeDtypeStruct((N, H), F32)),
    )(t, h2, agg, w, b.reshape(1, H), g.reshape(1, H), bt.reshape(1, H))


def _final_body(t_ref, h2_ref, agg_ref, w_ref, b_ref, g_ref, bt_ref,
                wp_ref, bp_ref, o_ref):
    agg = agg_ref[...][:N].astype(F32)
    t = _mm(h2_ref[...] + agg, w_ref[...]) + b_ref[...] + t_ref[...]
    h = _bn_relu(t, g_ref[...], bt_ref[...])
    o_ref[...] = _mm(h, wp_ref[...]) + bp_ref[...]


def _final(t, h2, agg, w, b, g, bt, wp, bp):
    return pl.pallas_call(
        _final_body,
        out_shape=jax.ShapeDtypeStruct((N, T), F32),
    )(t, h2, agg, w, b.reshape(1, H), g.reshape(1, H), bt.reshape(1, H),
      wp, bp.reshape(1, T))


def _sort_body(src_hbm, dst_hbm, srt_hbm, rp_hbm,
               cnt_my, off_my, dbuf, sbuf, posb, cbuf, csh, srt_sh, sem):
    c = lax.axis_index("c")
    s = lax.axis_index("s")
    base_e = c * EH + s * EWS

    def zero_body(i, _):
        cnt_my[pl.ds(i * 16, 16)] = jnp.zeros((16,), I32)
        return 0
    lax.fori_loop(0, NN // 16, zero_body, 0)

    for k in range(EWS // CHK):
        pltpu.sync_copy(dst_hbm.at[pl.ds(base_e + k * CHK, CHK)],
                        dbuf.at[pl.ds(0, CHK)])

        def hist_body(i, _):
            v = dbuf[pl.ds(i * 16, 16)]
            cnt, last = plsc.scan_count(v)
            cur = plsc.load_gather(cnt_my, [v])
            plsc.store_scatter(cnt_my, [v], cur + cnt, mask=last)
            return 0
        lax.fori_loop(0, NVC, hist_body, 0)

    pltpu.sync_copy(cnt_my, csh.at[s])
    plsc.subcore_barrier()

    def blk_body(blk, carry):
        cps = [pltpu.async_copy(csh.at[r, pl.ds(blk * BB, BB)], cbuf.at[r], sem)
               for r in range(NS)]
        for cp in cps:
            cp.wait()

        def vreg_body(j, carry):
            t = jnp.zeros((16,), I32)
            p = jnp.zeros((16,), I32)
            for r in range(NS):
                row = cbuf[r, pl.ds(j * 16, 16)]
                t = t + row
                p = p + jnp.where(r < s, row, jnp.zeros((16,), I32))
            incl = plsc.cumsum(t)
            excl = incl - t + carry
            cnt_my[pl.ds(blk * BB + j * 16, 16)] = excl
            off_my[pl.ds(blk * BB + j * 16, 16)] = excl + p
            return carry + jnp.max(incl, axis=0)
        return lax.fori_loop(0, BB // 16, vreg_body, carry)
    lax.fori_loop(0, NN // BB, blk_body, jnp.zeros((), I32))

    nb = NN // NS
    pltpu.sync_copy(cnt_my.at[pl.ds(s * nb, nb)],
                    rp_hbm.at[pl.ds(c * RPS + s * nb, nb)])

    @pl.when(s == NS - 1)
    def _():
        dbuf[pl.ds(0, 16)] = jnp.full((16,), EH, I32)
        pltpu.sync_copy(dbuf.at[pl.ds(0, 16)], rp_hbm.at[pl.ds(c * RPS + NN, 16)])

    dump = EH + 8 + s * 8
    for k in range(EWS // CHK):
        pltpu.sync_copy(dst_hbm.at[pl.ds(base_e + k * CHK, CHK)],
                        dbuf.at[pl.ds(0, CHK)])
        pltpu.sync_copy(src_hbm.at[pl.ds(base_e + k * CHK, CHK)],
                        sbuf.at[pl.ds(0, CHK)])
        for j in range(16):
            for u in range(8):
                eb = j * SB + u * 16
                if eb < CHK:
                    v = dbuf[pl.ds(eb, 16)]
                    cnt, last = plsc.scan_count(v)
                    off = plsc.load_gather(off_my, [v])
                    posb[0, j, pl.ds(u * 16, 16)] = off + cnt - 1
                    plsc.store_scatter(off_my, [v], off + cnt, mask=last)
                else:
                    posb[0, j, pl.ds(u * 16, 16)] = jnp.full((16,), dump, I32)
        copies = []
        for j in range(16):
            copies.append(pltpu.async_copy(
                sbuf.at[pl.ds(j * SB, SB)], srt_sh.at[posb.at[0, j]], sem))
        for cp in copies:
            cp.wait()

    plsc.subcore_barrier()
    ews = EH // NS
    pltpu.sync_copy(srt_sh.at[pl.ds(s * ews, ews)],
                    cnt_my.at[pl.ds(0, ews)])
    pltpu.sync_copy(cnt_my.at[pl.ds(0, ews)],
                    srt_hbm.at[pl.ds(c * ESH + s * ews, ews)])


def _sort_edges(src, dst):
    k = pl.kernel(
        _sort_body,
        out_type=(jax.ShapeDtypeStruct((NC * ESH,), I32),
                  jax.ShapeDtypeStruct((NC * RPS,), I32)),
        mesh=_SC_MESH,
        compiler_params=_CP,
        scratch_types=[
            pltpu.VMEM((NN,), I32),
            pltpu.VMEM((NN,), I32),
            pltpu.VMEM((CHK + 48,), I32),
            pltpu.VMEM((CHK + 48,), I32),
            pltpu.VMEM((1, 16, SB), I32),
            pltpu.VMEM((NS, BB), I32),
            pltpu.VMEM_SHARED((NS, NN), I32),
            pltpu.VMEM_SHARED((EH + 144,), I32),
            pltpu.SemaphoreType.DMA,
        ],
    )
    return k(src, dst)


def _segmax_body(m_hbm, srt_hbm, rp_hbm, agg_hbm,
                 aggb, rpb, ib, ib2, rows0, rows1, sem0, sem1):
    c = lax.axis_index("c")
    s = lax.axis_index("s")
    w = s * NC + c
    node_base = w * NBW


    def zrow(r, _):
        for u in range(8):
            aggb[r, pl.ds(u * 16, 16)] = jnp.zeros((16,), F32)
        return 0
    lax.fori_loop(0, NBW, zrow, 0)

    zacc = (jnp.zeros((16,), F32),) * 8

    for run in range(NC):
        rp_off = run * RPS + node_base
        pltpu.sync_copy(rp_hbm.at[pl.ds(rp_off, NBW + 16)],
                        rpb.at[pl.ds(0, NBW + 16)])

        def getrp(n):
            return rpb[pl.ds(n, 16)][0]

        e0 = getrp(0)
        e1 = getrp(NBW)
        count = e1 - e0

        def deg(n):
            return rpb[pl.ds(n + 1, 16)][0] - rpb[pl.ds(n, 16)][0]

        def adv_cond(cur):
            return jnp.logical_and(cur < NBW, deg(cur) == 0)

        def adv_body(cur):
            return cur + 1
        cur0 = lax.while_loop(adv_cond, adv_body, jnp.zeros((), I32))
        rem0 = jnp.where(cur0 < NBW, deg(jnp.minimum(cur0, NBW - 1)),
                         jnp.zeros((), I32))

        nchunks = (count + CE - 1) // CE

        def chunk_body(ck, st):
            cur, rem, acc = st
            start = e0 + ck * CE
            clen = jnp.minimum(CE, count - ck * CE)
            offa = (start // 8) * 8
            pad = start - offa
            pltpu.sync_copy(srt_hbm.at[pl.ds(run * ESH + offa, CE + 16)], ib)

            def clampb(i, _):
                v = ib[pl.ds(pad + i * 16, 16)]
                ib2[pl.ds(i * 16, 16)] = jnp.clip(v, 0, N - 1)
                return 0
            lax.fori_loop(0, CE // 16, clampb, 0)

            nb = (clen + GB - 1) // GB

            def fire(b, rbuf, rsem):
                return pltpu.async_copy(
                    m_hbm.at[ib2.at[pl.ds(b * GB, GB)]], rbuf, rsem)

            cp0 = fire(jnp.zeros((), I32), rows0, sem0)

            def batch_body(b, st):
                even = (b % 2) == 0

                @pl.when(jnp.logical_and(even, b + 1 < nb))
                def _():
                    fire(b + 1, rows1, sem1)

                @pl.when(jnp.logical_and(~even, b + 1 < nb))
                def _():
                    fire(b + 1, rows0, sem0)

                blen = jnp.minimum(GB, clen - b * GB)

                def proc(rbuf, rsem, st):
                    pltpu.make_async_copy(
                        m_hbm.at[ib2.at[pl.ds(0, GB)]], rbuf, rsem).wait()

                    def piece_cond(pst):
                        e, cur, rem, acc = pst
                        return jnp.logical_and(e < blen, cur < NBW)

                    def piece_body(pst):
                        e, cur, rem, acc = pst
                        t = jnp.minimum(rem, blen - e)

                        def maxbody(i, acc):
                            return tuple(
                                jnp.maximum(acc[u], rbuf[i, pl.ds(u * 16, 16)])
                                for u in range(8))
                        acc = lax.fori_loop(e, e + t, maxbody, acc)
                        rem = rem - t
                        e = e + t

                        def flush(op):
                            cur, acc = op
                            for u in range(8):
                                old = aggb[cur, pl.ds(u * 16, 16)]
                                aggb[cur, pl.ds(u * 16, 16)] = (
                                    jnp.maximum(old, acc[u]))
                            nxt = lax.while_loop(adv_cond, adv_body, cur + 1)
                            nrem = jnp.where(nxt < NBW,
                                             deg(jnp.minimum(nxt, NBW - 1)),
                                             jnp.zeros((), I32))
                            return nxt, nrem, zacc

                        def noflush(op):
                            cur, acc = op
                            return cur, rem, acc

                        cur, rem, acc = lax.cond(rem == 0, flush, noflush,
                                                 (cur, acc))
                        return e, cur, rem, acc

                    cur0b, rem0b, acc0b = st
                    _, cur1, rem1, acc1 = lax.while_loop(
                        piece_cond, piece_body,
                        (jnp.zeros((), I32), cur0b, rem0b, acc0b))
                    return cur1, rem1, acc1

                return lax.cond(even,
                                lambda st: proc(rows0, sem0, st),
                                lambda st: proc(rows1, sem1, st), st)

            cur, rem, acc = lax.fori_loop(0, nb, batch_body, (cur, rem, acc))
            return cur, rem, acc

        lax.fori_loop(0, nchunks, chunk_body, (cur0, rem0, zacc))

    pltpu.sync_copy(aggb, agg_hbm.at[pl.ds(node_base, NBW)])


def _segmax(m, srt, rp):
    k = pl.kernel(
        _segmax_body,
        out_type=jax.ShapeDtypeStruct((NN, H), F32),
        mesh=_SC_MESH,
        compiler_params=_CP,
        scratch_types=[
            pltpu.VMEM((NBW, H), F32),
            pltpu.VMEM((NBW + 32,), I32),
            pltpu.VMEM((CE + 16,), I32),
            pltpu.VMEM((CE + 16,), I32),
            pltpu.VMEM((GB, H), F32),
            pltpu.VMEM((GB, H), F32),
            pltpu.SemaphoreType.DMA,
            pltpu.SemaphoreType.DMA,
        ],
    )
    return k(m, srt, rp)


def kernel(x, edge_index, batch, W_enc, b_enc, gcn_W, gcn_b, bn_gamma,
           bn_beta, W_pred, b_pred):
    src = edge_index[0]
    dst = edge_index[1]
    srt, rp = _sort_edges(src, dst)
    h0, m0 = _encode(x, W_enc, b_enc)
    a0 = _segmax(m0, srt, rp)
    t1, h2_1, m1 = _layer(h0, h0, a0, gcn_W[0], gcn_b[0],
                          bn_gamma[0], bn_beta[0], has_res=False)
    a1 = _segmax(m1, srt, rp)
    t2, h2_2, m2 = _layer(t1, h2_1, a1, gcn_W[1], gcn_b[1],
                          bn_gamma[1], bn_beta[1], has_res=True)
    a2 = _segmax(m2, srt, rp)
    return _final(t2, h2_2, a2, gcn_W[2], gcn_b[2], bn_gamma[2], bn_beta[2],
                  W_pred, b_pred)
